# Initial kernel scaffold; baseline (speedup 1.0000x reference)
import jax
import jax.numpy as jnp
from jax.experimental import pallas as pl


def kernel(g, node_feat, edge_weight, hidden_state, Wz, bz, Lz_w, Lz_b, Wr, br, Lr_w, Lr_b, Wh, bh, Lh_w, Lh_b, W1, b1, W2, b2):
    raise NotImplementedError("write your pallas kernel here")


# speedup vs baseline: 12.9342x; 12.9342x over previous
import functools

import jax
import jax.numpy as jnp
from jax import lax
from jax.experimental import pallas as pl
from jax.experimental.pallas import tpu as pltpu
from jax.experimental.pallas import tpu_sc as plsc

N = 10000
E = 320000
D = 128
NPAD = 10240
CHUNK = 128
NS = 16
NC = 2
NCHUNKS = 2560
EPAD = NCHUNKS * CHUNK
DEG_PER_TILE = NCHUNKS // NS
MSG_PER_W = NCHUNKS // (NS * NC)
BLOCKC = 16
ROWS_PER_TILE = NPAD // NS


def _sc_body(row_hbm, col_hbm, w_hbm, g_hbm, out_hbm,
             col_st, w_st, row_st, dinv, normb, rows, deg_sh, p_sh, sem):
    c = lax.axis_index("c")
    s = lax.axis_index("s")
    w_id = c * NS + s

    zero16 = jnp.zeros((16,), jnp.float32)

    def zrow(e, carry):
        for u in range(8):
            rows[e, pl.ds(u * 16, 16)] = zero16
        return carry
    lax.fori_loop(0, 128, zrow, 0)
    for u in range(8):
        normb[pl.ds(u * 16, 16)] = zero16

    for k in range(5):
        pltpu.sync_copy(normb, deg_sh.at[pl.ds(s * 640 + k * 128, 128)])
    for k in range(5):
        pltpu.sync_copy(rows, p_sh.at[pl.ds(s * ROWS_PER_TILE + k * 128, 128)])

    plsc.subcore_barrier()

    def deg_blk(b, carry):
        base = s * DEG_PER_TILE + b * BLOCKC
        pltpu.sync_copy(col_hbm.at[pl.ds(base, BLOCKC)], col_st)
        pltpu.sync_copy(w_hbm.at[pl.ds(base, BLOCKC)], w_st)

        def deg_chunk(j, c2):
            pltpu.sync_copy(w_st.at[j], deg_sh.at[col_st.at[j]], add=True)
            return c2
        lax.fori_loop(0, BLOCKC, deg_chunk, 0)
        return carry
    lax.fori_loop(0, DEG_PER_TILE // BLOCKC, deg_blk, 0)

    plsc.subcore_barrier()

    seg = pl.ds(s * 640, 640)
    pltpu.sync_copy(deg_sh.at[seg], dinv.at[pl.ds(0, 640)])

    def bab(i, carry):
        x = dinv[pl.ds(i * 16, 16)]
        m = x > 0.0
        xs = jnp.maximum(jnp.where(m, x, 1.0), 1e-9)
        d = 1.0 / xs
        sq = (1.0 + d) * 0.5
        for _ in range(20):
            sq = 0.5 * (sq + d / sq)
        dinv[pl.ds(i * 16, 16)] = jnp.where(m, sq, 0.0)
        return carry
    lax.fori_loop(0, 40, bab, 0)

    pltpu.sync_copy(dinv.at[pl.ds(0, 640)], deg_sh.at[seg])
    plsc.subcore_barrier()
    pltpu.sync_copy(deg_sh, dinv)

    def do_chunk(j):
        def normi(i, carry):
            r16 = row_st[j, pl.ds(i * 16, 16)]
            c16 = col_st[j, pl.ds(i * 16, 16)]
            w16 = w_st[j, pl.ds(i * 16, 16)]
            dr = plsc.load_gather(dinv, [r16])
            dc = plsc.load_gather(dinv, [c16])
            normb[pl.ds(i * 16, 16)] = dr * w16 * dc
            return carry
        lax.fori_loop(0, 8, normi, 0)

        pltpu.async_copy(g_hbm.at[row_st.at[j]], rows, sem).wait()

        def scale(gi, carry):
            n16 = normb[pl.ds(gi * 16, 16)]
            for l in range(16):
                nv = n16[l]
                e = gi * 16 + l
                for u in range(8):
                    sl = pl.ds(u * 16, 16)
                    rows[e, sl] = rows[e, sl] * nv
            return carry
        lax.fori_loop(0, 8, scale, 0)

        pltpu.sync_copy(rows, p_sh.at[col_st.at[j]], add=True)

    def msg_blk(b, carry):
        base = w_id * MSG_PER_W + b * BLOCKC
        pltpu.sync_copy(row_hbm.at[pl.ds(base, BLOCKC)], row_st)
        pltpu.sync_copy(col_hbm.at[pl.ds(base, BLOCKC)], col_st)
        pltpu.sync_copy(w_hbm.at[pl.ds(base, BLOCKC)], w_st)

        def msg_chunk(j, c2):
            do_chunk(j)
            return c2
        lax.fori_loop(0, BLOCKC, msg_chunk, 0)
        return carry
    lax.fori_loop(0, MSG_PER_W // BLOCKC, msg_blk, 0)

    plsc.subcore_barrier()

    ob = s * ROWS_PER_TILE
    pltpu.sync_copy(p_sh.at[pl.ds(ob, ROWS_PER_TILE)],
                    out_hbm.at[c, pl.ds(ob, ROWS_PER_TILE)])


_sc_sparse = functools.partial(
    pl.kernel,
    out_type=jax.ShapeDtypeStruct((NC, NPAD, D), jnp.float32),
    mesh=plsc.VectorSubcoreMesh(core_axis_name="c", subcore_axis_name="s"),
    compiler_params=pltpu.CompilerParams(needs_layout_passes=False),
    scratch_types=[
        pltpu.VMEM((BLOCKC, CHUNK), jnp.int32),
        pltpu.VMEM((BLOCKC, CHUNK), jnp.float32),
        pltpu.VMEM((BLOCKC, CHUNK), jnp.int32),
        pltpu.VMEM((NPAD,), jnp.float32),
        pltpu.VMEM((CHUNK,), jnp.float32),
        pltpu.VMEM((CHUNK, D), jnp.float32),
        pltpu.VMEM_SHARED((NPAD,), jnp.float32),
        pltpu.VMEM_SHARED((NPAD, D), jnp.float32),
        pltpu.SemaphoreType.DMA,
    ],
)(_sc_body)


BLK = 256
_GRID = (N + BLK - 1) // BLK


def _dense_body(p0, p1, h0, Wz, Wr, Wh, Lz, Lr, Lh, W1, W2,
                bz, br, bh, Lzb, Lrb, Lhb, b1, b2,
                yo_ref, y1_ref, hn_ref):
    f32 = jnp.float32

    def dot(a, b):
        return lax.dot(a, b, precision=lax.Precision.HIGHEST,
                       preferred_element_type=f32)

    P = p0[...] + p1[...]
    h = h0[...]
    cz = dot(P, Wz[...]) + bz[...]
    cr = dot(P, Wr[...]) + br[...]
    ch = dot(P, Wh[...]) + bh[...]
    Lzm, Lrm, Lhm = Lz[...], Lr[...], Lh[...]
    Z = jax.nn.sigmoid(dot(cz, Lzm[:D]) + dot(h, Lzm[D:]) + Lzb[...])
    R = jax.nn.sigmoid(dot(cr, Lrm[:D]) + dot(h, Lrm[D:]) + Lrb[...])
    Ht = jnp.tanh(dot(ch, Lhm[:D]) + dot(h * R, Lhm[D:]) + Lhb[...])
    hn = Z * h + (1.0 - Z) * Ht
    y1 = dot(jnp.maximum(hn, 0.0), W1[...]) + b1[...]
    yo = dot(y1, W2[...]) + b2[...]
    yo_ref[...] = yo
    y1_ref[...] = y1
    hn_ref[...] = hn


def _dense(p0, p1, h0, Wz, Wr, Wh, Lz, Lr, Lh, W1, W2,
           bz, br, bh, Lzb, Lrb, Lhb, b1, b2):
    rowspec = pl.BlockSpec((BLK, D), lambda i: (i, 0))
    wspec = pl.BlockSpec((D, D), lambda i: (0, 0))
    lspec = pl.BlockSpec((2 * D, D), lambda i: (0, 0))
    bspec = pl.BlockSpec((1, D), lambda i: (0, 0))
    out_shape = jax.ShapeDtypeStruct((N, D), jnp.float32)
    return pl.pallas_call(
        _dense_body,
        grid=(_GRID,),
        in_specs=[rowspec, rowspec, rowspec,
                  wspec, wspec, wspec, lspec, lspec, lspec, wspec, wspec,
                  bspec, bspec, bspec, bspec, bspec, bspec, bspec, bspec],
        out_specs=[rowspec, rowspec, rowspec],
        out_shape=[out_shape, out_shape, out_shape],
    )(p0, p1, h0, Wz, Wr, Wh, Lz, Lr, Lh, W1, W2,
      bz, br, bh, Lzb, Lrb, Lhb, b1, b2)


def kernel(g, node_feat, edge_weight, hidden_state,
           Wz, bz, Lz_w, Lz_b, Wr, br, Lr_w, Lr_b, Wh, bh, Lh_w, Lh_b,
           W1, b1, W2, b2):
    pad = EPAD - E
    row2d = jnp.concatenate(
        [node_feat[0], jnp.zeros((pad,), jnp.int32)]).reshape(NCHUNKS, CHUNK)
    col2d = jnp.concatenate(
        [node_feat[1], jnp.zeros((pad,), jnp.int32)]).reshape(NCHUNKS, CHUNK)
    w2d = jnp.concatenate(
        [edge_weight, jnp.zeros((pad,), jnp.float32)]).reshape(NCHUNKS, CHUNK)

    p_parts = _sc_sparse(row2d, col2d, w2d, g)

    r2 = lambda b: b.reshape(1, D)
    yo, y1, hn = _dense(p_parts[0], p_parts[1], hidden_state,
                        Wz, Wr, Wh, Lz_w, Lr_w, Lh_w, W1, W2,
                        r2(bz), r2(br), r2(bh), r2(Lz_b), r2(Lr_b), r2(Lh_b),
                        r2(b1), r2(b2))
    return (yo, y1, hn)


# speedup vs baseline: 23.9767x; 1.8537x over previous
import functools

import jax
import jax.numpy as jnp
from jax import lax
from jax.experimental import pallas as pl
from jax.experimental.pallas import tpu as pltpu
from jax.experimental.pallas import tpu_sc as plsc

N = 10000
E = 320000
D = 128
NPAD = 10240
CHUNK = 128
NS = 16
NC = 2
NCHUNKS = 2560
EPAD = NCHUNKS * CHUNK
DEG_PER_TILE = NCHUNKS // NS
MSG_PER_W = NCHUNKS // (NS * NC)
BLOCKC = 16
ROWS_PER_TILE = NPAD // NS


def _sc_body(row_hbm, col_hbm, w_hbm, g_hbm, out_hbm,
             col_st, w_st, row_st, dinv, normb, rows, deg_sh, p_sh, sem):
    c = lax.axis_index("c")
    s = lax.axis_index("s")
    w_id = c * NS + s

    zero16 = jnp.zeros((16,), jnp.float32)

    def zrow(e, carry):
        for u in range(8):
            rows[e, pl.ds(u * 16, 16)] = zero16
        return carry
    lax.fori_loop(0, 128, zrow, 0)
    for u in range(8):
        normb[pl.ds(u * 16, 16)] = zero16

    for k in range(5):
        pltpu.sync_copy(normb, deg_sh.at[pl.ds(s * 640 + k * 128, 128)])
    for k in range(5):
        pltpu.sync_copy(rows, p_sh.at[pl.ds(s * ROWS_PER_TILE + k * 128, 128)])

    plsc.subcore_barrier()

    def deg_blk(b, carry):
        base = s * DEG_PER_TILE + b * BLOCKC
        pltpu.sync_copy(col_hbm.at[pl.ds(base, BLOCKC)], col_st)
        pltpu.sync_copy(w_hbm.at[pl.ds(base, BLOCKC)], w_st)

        def deg_chunk(j, c2):
            pltpu.sync_copy(w_st.at[j], deg_sh.at[col_st.at[j]], add=True)
            return c2
        lax.fori_loop(0, BLOCKC, deg_chunk, 0)
        return carry
    lax.fori_loop(0, DEG_PER_TILE // BLOCKC, deg_blk, 0)

    plsc.subcore_barrier()

    seg = pl.ds(s * 640, 640)
    pltpu.sync_copy(deg_sh.at[seg], dinv.at[pl.ds(0, 640)])

    def bab(i, carry):
        x = dinv[pl.ds(i * 16, 16)]
        m = x > 0.0
        xs = jnp.maximum(jnp.where(m, x, 1.0), 1e-9)
        d = 1.0 / xs
        sq = (1.0 + d) * 0.5
        for _ in range(20):
            sq = 0.5 * (sq + d / sq)
        dinv[pl.ds(i * 16, 16)] = jnp.where(m, sq, 0.0)
        return carry
    lax.fori_loop(0, 40, bab, 0)

    pltpu.sync_copy(dinv.at[pl.ds(0, 640)], deg_sh.at[seg])
    plsc.subcore_barrier()
    pltpu.sync_copy(deg_sh, dinv)

    def do_chunk(j):
        def normi(i, carry):
            r16 = row_st[j, pl.ds(i * 16, 16)]
            c16 = col_st[j, pl.ds(i * 16, 16)]
            w16 = w_st[j, pl.ds(i * 16, 16)]
            dr = plsc.load_gather(dinv, [r16])
            dc = plsc.load_gather(dinv, [c16])
            normb[pl.ds(i * 16, 16)] = dr * w16 * dc
            return carry
        lax.fori_loop(0, 8, normi, 0)

        pltpu.async_copy(g_hbm.at[row_st.at[j]], rows, sem).wait()

        def scale(gi, carry):
            n16 = normb[pl.ds(gi * 16, 16)]
            for l in range(16):
                nv = n16[l]
                e = gi * 16 + l
                for u in range(8):
                    sl = pl.ds(u * 16, 16)
                    rows[e, sl] = rows[e, sl] * nv
            return carry
        lax.fori_loop(0, 8, scale, 0)

        pltpu.sync_copy(rows, p_sh.at[col_st.at[j]], add=True)

    def msg_blk(b, carry):
        base = w_id * MSG_PER_W + b * BLOCKC
        pltpu.sync_copy(row_hbm.at[pl.ds(base, BLOCKC)], row_st)
        pltpu.sync_copy(col_hbm.at[pl.ds(base, BLOCKC)], col_st)
        pltpu.sync_copy(w_hbm.at[pl.ds(base, BLOCKC)], w_st)

        def msg_chunk(j, c2):
            do_chunk(j)
            return c2
        lax.fori_loop(0, BLOCKC, msg_chunk, 0)
        return carry
    lax.fori_loop(0, MSG_PER_W // BLOCKC, msg_blk, 0)

    plsc.subcore_barrier()

    ob = s * ROWS_PER_TILE
    pltpu.sync_copy(p_sh.at[pl.ds(ob, ROWS_PER_TILE)],
                    out_hbm.at[c, pl.ds(ob, ROWS_PER_TILE)])


_sc_sparse = functools.partial(
    pl.kernel,
    out_type=jax.ShapeDtypeStruct((NC, NPAD, D), jnp.float32),
    mesh=plsc.VectorSubcoreMesh(core_axis_name="c", subcore_axis_name="s"),
    compiler_params=pltpu.CompilerParams(needs_layout_passes=False),
    scratch_types=[
        pltpu.VMEM((BLOCKC, CHUNK), jnp.int32),
        pltpu.VMEM((BLOCKC, CHUNK), jnp.float32),
        pltpu.VMEM((BLOCKC, CHUNK), jnp.int32),
        pltpu.VMEM((NPAD,), jnp.float32),
        pltpu.VMEM((CHUNK,), jnp.float32),
        pltpu.VMEM((CHUNK, D), jnp.float32),
        pltpu.VMEM_SHARED((NPAD,), jnp.float32),
        pltpu.VMEM_SHARED((NPAD, D), jnp.float32),
        pltpu.SemaphoreType.DMA,
    ],
)(_sc_body)


BLK = 256
_GRID = (N + BLK - 1) // BLK


def _dense_body(p0, p1, h0, Wz, Wr, Wh, Lz, Lr, Lh, W1, W2,
                bz, br, bh, Lzb, Lrb, Lhb, b1, b2,
                yo_ref, y1_ref, hn_ref):
    f32 = jnp.float32

    def dot(a, b):
        return lax.dot(a, b, precision=lax.Precision.HIGHEST,
                       preferred_element_type=f32)

    P = p0[...] + p1[...]
    h = h0[...]
    cz = dot(P, Wz[...]) + bz[...]
    cr = dot(P, Wr[...]) + br[...]
    ch = dot(P, Wh[...]) + bh[...]
    Lzm, Lrm, Lhm = Lz[...], Lr[...], Lh[...]
    Z = jax.nn.sigmoid(dot(cz, Lzm[:D]) + dot(h, Lzm[D:]) + Lzb[...])
    R = jax.nn.sigmoid(dot(cr, Lrm[:D]) + dot(h, Lrm[D:]) + Lrb[...])
    Ht = jnp.tanh(dot(ch, Lhm[:D]) + dot(h * R, Lhm[D:]) + Lhb[...])
    hn = Z * h + (1.0 - Z) * Ht
    y1 = dot(jnp.maximum(hn, 0.0), W1[...]) + b1[...]
    yo = dot(y1, W2[...]) + b2[...]
    yo_ref[...] = yo
    y1_ref[...] = y1
    hn_ref[...] = hn


def _dense(p0, p1, h0, Wz, Wr, Wh, Lz, Lr, Lh, W1, W2,
           bz, br, bh, Lzb, Lrb, Lhb, b1, b2):
    rowspec = pl.BlockSpec((BLK, D), lambda i: (i, 0))
    wspec = pl.BlockSpec((D, D), lambda i: (0, 0))
    lspec = pl.BlockSpec((2 * D, D), lambda i: (0, 0))
    bspec = pl.BlockSpec((1, D), lambda i: (0, 0))
    out_shape = jax.ShapeDtypeStruct((N, D), jnp.float32)
    return pl.pallas_call(
        _dense_body,
        grid=(_GRID,),
        in_specs=[rowspec, rowspec, rowspec,
                  wspec, wspec, wspec, lspec, lspec, lspec, wspec, wspec,
                  bspec, bspec, bspec, bspec, bspec, bspec, bspec, bspec],
        out_specs=[rowspec, rowspec, rowspec],
        out_shape=[out_shape, out_shape, out_shape],
    )(p0, p1, h0, Wz, Wr, Wh, Lz, Lr, Lh, W1, W2,
      bz, br, bh, Lzb, Lrb, Lhb, b1, b2)


def kernel(g, node_feat, edge_weight, hidden_state,
           Wz, bz, Lz_w, Lz_b, Wr, br, Lr_w, Lr_b, Wh, bh, Lh_w, Lh_b,
           W1, b1, W2, b2):
    pad = EPAD - E
    pad_iota = jnp.arange(pad, dtype=jnp.int32)
    row2d = jnp.concatenate(
        [node_feat[0], pad_iota % N]).reshape(NCHUNKS, CHUNK)
    col2d = jnp.concatenate(
        [node_feat[1], N + pad_iota % (NPAD - N)]).reshape(NCHUNKS, CHUNK)
    w2d = jnp.concatenate(
        [edge_weight, jnp.zeros((pad,), jnp.float32)]).reshape(NCHUNKS, CHUNK)

    p_parts = _sc_sparse(row2d, col2d, w2d, g)

    r2 = lambda b: b.reshape(1, D)
    yo, y1, hn = _dense(p_parts[0], p_parts[1], hidden_state,
                        Wz, Wr, Wh, Lz_w, Lr_w, Lh_w, W1, W2,
                        r2(bz), r2(br), r2(bh), r2(Lz_b), r2(Lr_b), r2(Lh_b),
                        r2(b1), r2(b2))
    return (yo, y1, hn)


# speedup vs baseline: 24.3820x; 1.0169x over previous
import functools

import jax
import jax.numpy as jnp
from jax import lax
from jax.experimental import pallas as pl
from jax.experimental.pallas import tpu as pltpu
from jax.experimental.pallas import tpu_sc as plsc

N = 10000
E = 320000
D = 128
NPAD = 10240
CHUNK = 128
NS = 16
NC = 2
NCHUNKS = 2560
EPAD = NCHUNKS * CHUNK
DEG_PER_TILE = NCHUNKS // NS
MSG_PER_W = NCHUNKS // (NS * NC)
BLOCKC = 16
ROWS_PER_TILE = NPAD // NS


def _sc_body(row_hbm, col_hbm, w_hbm, g_hbm, out_hbm,
             col_st, w_st, row_st, dinv, normb, rows, deg_sh, p_sh, sem):
    c = lax.axis_index("c")
    s = lax.axis_index("s")
    w_id = c * NS + s

    zero16 = jnp.zeros((16,), jnp.float32)

    def zrow(e, carry):
        for u in range(8):
            rows[e, pl.ds(u * 16, 16)] = zero16
        return carry
    lax.fori_loop(0, 128, zrow, 0)
    for u in range(8):
        normb[pl.ds(u * 16, 16)] = zero16

    for k in range(5):
        pltpu.sync_copy(normb, deg_sh.at[pl.ds(s * 640 + k * 128, 128)])
    for k in range(5):
        pltpu.sync_copy(rows, p_sh.at[pl.ds(s * ROWS_PER_TILE + k * 128, 128)])

    plsc.subcore_barrier()

    def deg_blk(b, carry):
        base = s * DEG_PER_TILE + b * BLOCKC
        pltpu.sync_copy(col_hbm.at[pl.ds(base, BLOCKC)], col_st)
        pltpu.sync_copy(w_hbm.at[pl.ds(base, BLOCKC)], w_st)

        def deg_chunk(j, c2):
            pltpu.sync_copy(w_st.at[j], deg_sh.at[col_st.at[j]], add=True)
            return c2
        lax.fori_loop(0, BLOCKC, deg_chunk, 0)
        return carry
    lax.fori_loop(0, DEG_PER_TILE // BLOCKC, deg_blk, 0)

    plsc.subcore_barrier()

    seg = pl.ds(s * 640, 640)
    pltpu.sync_copy(deg_sh.at[seg], dinv.at[pl.ds(0, 640)])

    def bab(i, carry):
        x = dinv[pl.ds(i * 16, 16)]
        m = x > 0.0
        xs = jnp.maximum(jnp.where(m, x, 1.0), 1e-9)
        d = 1.0 / xs
        sq = (1.0 + d) * 0.5
        for _ in range(20):
            sq = 0.5 * (sq + d / sq)
        dinv[pl.ds(i * 16, 16)] = jnp.where(m, sq, 0.0)
        return carry
    lax.fori_loop(0, 40, bab, 0)

    pltpu.sync_copy(dinv.at[pl.ds(0, 640)], deg_sh.at[seg])
    plsc.subcore_barrier()
    pltpu.sync_copy(deg_sh, dinv)

    def do_chunk(j):
        cp = pltpu.async_copy(g_hbm.at[row_st.at[j]], rows, sem)

        def normi(i, carry):
            r16 = row_st[j, pl.ds(i * 16, 16)]
            c16 = col_st[j, pl.ds(i * 16, 16)]
            w16 = w_st[j, pl.ds(i * 16, 16)]
            dr = plsc.load_gather(dinv, [r16])
            dc = plsc.load_gather(dinv, [c16])
            normb[pl.ds(i * 16, 16)] = dr * w16 * dc
            return carry
        lax.fori_loop(0, 8, normi, 0)

        cp.wait()

        def scale(gi, carry):
            n16 = normb[pl.ds(gi * 16, 16)]
            for l in range(16):
                nv = n16[l]
                e = gi * 16 + l
                for u in range(8):
                    sl = pl.ds(u * 16, 16)
                    rows[e, sl] = rows[e, sl] * nv
            return carry
        lax.fori_loop(0, 8, scale, 0)

        pltpu.sync_copy(rows, p_sh.at[col_st.at[j]], add=True)

    def msg_blk(b, carry):
        base = w_id * MSG_PER_W + b * BLOCKC
        pltpu.sync_copy(row_hbm.at[pl.ds(base, BLOCKC)], row_st)
        pltpu.sync_copy(col_hbm.at[pl.ds(base, BLOCKC)], col_st)
        pltpu.sync_copy(w_hbm.at[pl.ds(base, BLOCKC)], w_st)

        def msg_chunk(j, c2):
            do_chunk(j)
            return c2
        lax.fori_loop(0, BLOCKC, msg_chunk, 0)
        return carry
    lax.fori_loop(0, MSG_PER_W // BLOCKC, msg_blk, 0)

    plsc.subcore_barrier()

    ob = s * ROWS_PER_TILE
    pltpu.sync_copy(p_sh.at[pl.ds(ob, ROWS_PER_TILE)],
                    out_hbm.at[c, pl.ds(ob, ROWS_PER_TILE)])


_sc_sparse = functools.partial(
    pl.kernel,
    out_type=jax.ShapeDtypeStruct((NC, NPAD, D), jnp.float32),
    mesh=plsc.VectorSubcoreMesh(core_axis_name="c", subcore_axis_name="s"),
    compiler_params=pltpu.CompilerParams(needs_layout_passes=False),
    scratch_types=[
        pltpu.VMEM((BLOCKC, CHUNK), jnp.int32),
        pltpu.VMEM((BLOCKC, CHUNK), jnp.float32),
        pltpu.VMEM((BLOCKC, CHUNK), jnp.int32),
        pltpu.VMEM((NPAD,), jnp.float32),
        pltpu.VMEM((CHUNK,), jnp.float32),
        pltpu.VMEM((CHUNK, D), jnp.float32),
        pltpu.VMEM_SHARED((NPAD,), jnp.float32),
        pltpu.VMEM_SHARED((NPAD, D), jnp.float32),
        pltpu.SemaphoreType.DMA,
    ],
)(_sc_body)


BLK = 256
_GRID = (N + BLK - 1) // BLK


def _dense_body(p0, p1, h0, Wz, Wr, Wh, Lz, Lr, Lh, W1, W2,
                bz, br, bh, Lzb, Lrb, Lhb, b1, b2,
                yo_ref, y1_ref, hn_ref):
    f32 = jnp.float32

    def dot(a, b):
        return lax.dot(a, b, precision=lax.Precision.HIGHEST,
                       preferred_element_type=f32)

    P = p0[...] + p1[...]
    h = h0[...]
    cz = dot(P, Wz[...]) + bz[...]
    cr = dot(P, Wr[...]) + br[...]
    ch = dot(P, Wh[...]) + bh[...]
    Lzm, Lrm, Lhm = Lz[...], Lr[...], Lh[...]
    Z = jax.nn.sigmoid(dot(cz, Lzm[:D]) + dot(h, Lzm[D:]) + Lzb[...])
    R = jax.nn.sigmoid(dot(cr, Lrm[:D]) + dot(h, Lrm[D:]) + Lrb[...])
    Ht = jnp.tanh(dot(ch, Lhm[:D]) + dot(h * R, Lhm[D:]) + Lhb[...])
    hn = Z * h + (1.0 - Z) * Ht
    y1 = dot(jnp.maximum(hn, 0.0), W1[...]) + b1[...]
    yo = dot(y1, W2[...]) + b2[...]
    yo_ref[...] = yo
    y1_ref[...] = y1
    hn_ref[...] = hn


def _dense(p0, p1, h0, Wz, Wr, Wh, Lz, Lr, Lh, W1, W2,
           bz, br, bh, Lzb, Lrb, Lhb, b1, b2):
    rowspec = pl.BlockSpec((BLK, D), lambda i: (i, 0))
    wspec = pl.BlockSpec((D, D), lambda i: (0, 0))
    lspec = pl.BlockSpec((2 * D, D), lambda i: (0, 0))
    bspec = pl.BlockSpec((1, D), lambda i: (0, 0))
    out_shape = jax.ShapeDtypeStruct((N, D), jnp.float32)
    return pl.pallas_call(
        _dense_body,
        grid=(_GRID,),
        in_specs=[rowspec, rowspec, rowspec,
                  wspec, wspec, wspec, lspec, lspec, lspec, wspec, wspec,
                  bspec, bspec, bspec, bspec, bspec, bspec, bspec, bspec],
        out_specs=[rowspec, rowspec, rowspec],
        out_shape=[out_shape, out_shape, out_shape],
    )(p0, p1, h0, Wz, Wr, Wh, Lz, Lr, Lh, W1, W2,
      bz, br, bh, Lzb, Lrb, Lhb, b1, b2)


def kernel(g, node_feat, edge_weight, hidden_state,
           Wz, bz, Lz_w, Lz_b, Wr, br, Lr_w, Lr_b, Wh, bh, Lh_w, Lh_b,
           W1, b1, W2, b2):
    pad = EPAD - E
    pad_iota = jnp.arange(pad, dtype=jnp.int32)
    row2d = jnp.concatenate(
        [node_feat[0], pad_iota % N]).reshape(NCHUNKS, CHUNK)
    col2d = jnp.concatenate(
        [node_feat[1], N + pad_iota % (NPAD - N)]).reshape(NCHUNKS, CHUNK)
    w2d = jnp.concatenate(
        [edge_weight, jnp.zeros((pad,), jnp.float32)]).reshape(NCHUNKS, CHUNK)

    p_parts = _sc_sparse(row2d, col2d, w2d, g)

    r2 = lambda b: b.reshape(1, D)
    yo, y1, hn = _dense(p_parts[0], p_parts[1], hidden_state,
                        Wz, Wr, Wh, Lz_w, Lr_w, Lh_w, W1, W2,
                        r2(bz), r2(br), r2(bh), r2(Lz_b), r2(Lr_b), r2(Lh_b),
                        r2(b1), r2(b2))
    return (yo, y1, hn)


# speedup vs baseline: 28.3102x; 1.1611x over previous
import functools

import jax
import jax.numpy as jnp
import numpy as np
from jax import lax
from jax.experimental import pallas as pl
from jax.experimental.pallas import tpu as pltpu
from jax.experimental.pallas import tpu_sc as plsc

N = 10000
E = 320000
D = 128
NPAD = 10240
CHUNK = 128
NS = 16
NC = 2
NCHUNKS = 2560
EPAD = NCHUNKS * CHUNK
DEG_PER_TILE = NCHUNKS // NS
MSG_PER_W = NCHUNKS // (NS * NC)
BLOCKC = 16
ROWS_PER_TILE = NPAD // NS


def _sc_body(row_hbm, col_hbm, w_hbm, g_hbm, out_hbm,
             col_st, w_st, row_st, dinv, normb, rows, deg_sh, p_sh, sem):
    c = lax.axis_index("c")
    s = lax.axis_index("s")
    w_id = c * NS + s

    zero16 = jnp.zeros((16,), jnp.float32)

    def zrow(e, carry):
        for u in range(8):
            rows[e, pl.ds(u * 16, 16)] = zero16
        return carry
    lax.fori_loop(0, 128, zrow, 0)
    for u in range(8):
        normb[pl.ds(u * 16, 16)] = zero16

    for k in range(5):
        pltpu.sync_copy(normb, deg_sh.at[pl.ds(s * 640 + k * 128, 128)])
    for k in range(5):
        pltpu.sync_copy(rows, p_sh.at[pl.ds(s * ROWS_PER_TILE + k * 128, 128)])

    plsc.subcore_barrier()

    def deg_blk(b, carry):
        base = s * DEG_PER_TILE + b * BLOCKC
        pltpu.sync_copy(col_hbm.at[pl.ds(base, BLOCKC)], col_st)
        pltpu.sync_copy(w_hbm.at[pl.ds(base, BLOCKC)], w_st)

        def deg_chunk(j, c2):
            pltpu.sync_copy(w_st.at[j], deg_sh.at[col_st.at[j]], add=True)
            return c2
        lax.fori_loop(0, BLOCKC, deg_chunk, 0)
        return carry
    lax.fori_loop(0, DEG_PER_TILE // BLOCKC, deg_blk, 0)

    plsc.subcore_barrier()

    seg = pl.ds(s * 640, 640)
    pltpu.sync_copy(deg_sh.at[seg], dinv.at[pl.ds(0, 640)])

    def bab(i, carry):
        x = dinv[pl.ds(i * 16, 16)]
        m = x > 0.0
        xs = jnp.maximum(jnp.where(m, x, 1.0), 1e-9)
        d = 1.0 / xs
        sq = (1.0 + d) * 0.5
        for _ in range(20):
            sq = 0.5 * (sq + d / sq)
        dinv[pl.ds(i * 16, 16)] = jnp.where(m, sq, 0.0)
        return carry
    lax.fori_loop(0, 40, bab, 0)

    pltpu.sync_copy(dinv.at[pl.ds(0, 640)], deg_sh.at[seg])
    plsc.subcore_barrier()
    pltpu.sync_copy(deg_sh, dinv)

    def do_chunk(j):
        cp = pltpu.async_copy(g_hbm.at[row_st.at[j]], rows, sem)

        def normi(i, carry):
            r16 = row_st[j, pl.ds(i * 16, 16)]
            c16 = col_st[j, pl.ds(i * 16, 16)]
            w16 = w_st[j, pl.ds(i * 16, 16)]
            dr = plsc.load_gather(dinv, [r16])
            dc = plsc.load_gather(dinv, [c16])
            normb[pl.ds(i * 16, 16)] = dr * w16 * dc
            return carry
        lax.fori_loop(0, 8, normi, 0)

        cp.wait()

        def scale(gi, carry):
            n16 = normb[pl.ds(gi * 16, 16)]
            for l in range(16):
                nv = n16[l]
                e = gi * 16 + l
                for u in range(8):
                    sl = pl.ds(u * 16, 16)
                    rows[e, sl] = rows[e, sl] * nv
            return carry
        lax.fori_loop(0, 8, scale, 0)

        pltpu.sync_copy(rows, p_sh.at[col_st.at[j]], add=True)

    def msg_blk(b, carry):
        base = w_id * MSG_PER_W + b * BLOCKC
        pltpu.sync_copy(row_hbm.at[pl.ds(base, BLOCKC)], row_st)
        pltpu.sync_copy(col_hbm.at[pl.ds(base, BLOCKC)], col_st)
        pltpu.sync_copy(w_hbm.at[pl.ds(base, BLOCKC)], w_st)

        def msg_chunk(j, c2):
            do_chunk(j)
            return c2
        lax.fori_loop(0, BLOCKC, msg_chunk, 0)
        return carry
    lax.fori_loop(0, MSG_PER_W // BLOCKC, msg_blk, 0)

    plsc.subcore_barrier()

    ob = s * ROWS_PER_TILE
    pltpu.sync_copy(p_sh.at[pl.ds(ob, ROWS_PER_TILE)],
                    out_hbm.at[c, pl.ds(ob, ROWS_PER_TILE)])


_sc_sparse = functools.partial(
    pl.kernel,
    out_type=jax.ShapeDtypeStruct((NC, NPAD, D), jnp.float32),
    mesh=plsc.VectorSubcoreMesh(core_axis_name="c", subcore_axis_name="s"),
    compiler_params=pltpu.CompilerParams(needs_layout_passes=False),
    scratch_types=[
        pltpu.VMEM((BLOCKC, CHUNK), jnp.int32),
        pltpu.VMEM((BLOCKC, CHUNK), jnp.float32),
        pltpu.VMEM((BLOCKC, CHUNK), jnp.int32),
        pltpu.VMEM((NPAD,), jnp.float32),
        pltpu.VMEM((CHUNK,), jnp.float32),
        pltpu.VMEM((CHUNK, D), jnp.float32),
        pltpu.VMEM_SHARED((NPAD,), jnp.float32),
        pltpu.VMEM_SHARED((NPAD, D), jnp.float32),
        pltpu.SemaphoreType.DMA,
    ],
)(_sc_body)


BLK = 256
_GRID = (N + BLK - 1) // BLK


def _dense_body(p0, p1, h0, Wz, Wr, Wh, Lz, Lr, Lh, W1, W2,
                bz, br, bh, Lzb, Lrb, Lhb, b1, b2,
                yo_ref, y1_ref, hn_ref):
    f32 = jnp.float32

    def dot(a, b):
        return lax.dot(a, b, preferred_element_type=f32)

    P = p0[0] + p1[0]
    h = h0[...]
    cz = dot(P, Wz[...]) + bz[...]
    cr = dot(P, Wr[...]) + br[...]
    ch = dot(P, Wh[...]) + bh[...]
    Lzm, Lrm, Lhm = Lz[...], Lr[...], Lh[...]
    Z = jax.nn.sigmoid(dot(cz, Lzm[:D]) + dot(h, Lzm[D:]) + Lzb[...])
    R = jax.nn.sigmoid(dot(cr, Lrm[:D]) + dot(h, Lrm[D:]) + Lrb[...])
    Ht = jnp.tanh(dot(ch, Lhm[:D]) + dot(h * R, Lhm[D:]) + Lhb[...])
    hn = Z * h + (1.0 - Z) * Ht
    y1 = dot(jnp.maximum(hn, 0.0), W1[...]) + b1[...]
    yo = dot(y1, W2[...]) + b2[...]
    yo_ref[...] = yo
    y1_ref[...] = y1
    hn_ref[...] = hn


def _dense(pp, h0, Wz, Wr, Wh, Lz, Lr, Lh, W1, W2,
           bz, br, bh, Lzb, Lrb, Lhb, b1, b2):
    rowspec = pl.BlockSpec((BLK, D), lambda i: (i, 0))
    p0spec = pl.BlockSpec((1, BLK, D), lambda i: (0, i, 0))
    p1spec = pl.BlockSpec((1, BLK, D), lambda i: (1, i, 0))
    wspec = pl.BlockSpec((D, D), lambda i: (0, 0))
    lspec = pl.BlockSpec((2 * D, D), lambda i: (0, 0))
    bspec = pl.BlockSpec((1, D), lambda i: (0, 0))
    out_shape = jax.ShapeDtypeStruct((N, D), jnp.float32)
    return pl.pallas_call(
        _dense_body,
        grid=(_GRID,),
        in_specs=[p0spec, p1spec, rowspec,
                  wspec, wspec, wspec, lspec, lspec, lspec, wspec, wspec,
                  bspec, bspec, bspec, bspec, bspec, bspec, bspec, bspec],
        out_specs=[rowspec, rowspec, rowspec],
        out_shape=[out_shape, out_shape, out_shape],
    )(pp, pp, h0, Wz, Wr, Wh, Lz, Lr, Lh, W1, W2,
      bz, br, bh, Lzb, Lrb, Lhb, b1, b2)


def kernel(g, node_feat, edge_weight, hidden_state,
           Wz, bz, Lz_w, Lz_b, Wr, br, Lr_w, Lr_b, Wh, bh, Lh_w, Lh_b,
           W1, b1, W2, b2):
    pad = EPAD - E
    pad_iota = np.arange(pad, dtype=np.int32)
    pad_rows = jnp.asarray(pad_iota % N)
    pad_cols = jnp.asarray(N + pad_iota % (NPAD - N))
    pad_w = jnp.zeros((pad,), jnp.float32)
    row2d = jnp.concatenate([node_feat[0], pad_rows]).reshape(NCHUNKS, CHUNK)
    col2d = jnp.concatenate([node_feat[1], pad_cols]).reshape(NCHUNKS, CHUNK)
    w2d = jnp.concatenate([edge_weight, pad_w]).reshape(NCHUNKS, CHUNK)

    p_parts = _sc_sparse(row2d, col2d, w2d, g)

    r2 = lambda b: b.reshape(1, D)
    yo, y1, hn = _dense(p_parts, hidden_state,
                        Wz, Wr, Wh, Lz_w, Lr_w, Lh_w, W1, W2,
                        r2(bz), r2(br), r2(bh), r2(Lz_b), r2(Lr_b), r2(Lh_b),
                        r2(b1), r2(b2))
    return (yo, y1, hn)
